# baseline (speedup 1.0000x reference)
import functools
import math

import jax
import jax.numpy as jnp
from jax import lax
from jax.experimental import pallas as pl
from jax.experimental.pallas import tpu as pltpu
from jax.experimental.pallas import tpu_sc as plsc

_INFO = plsc.get_sparse_core_info()
_NC = _INFO.num_cores
_NS = _INFO.num_subcores
_NW = _NC * _NS
_L = _INFO.num_lanes

_V = 1000000
_D = 64
_SCALE = math.sqrt(_D)
_NBLK = (_V // 128)
_VTAIL = _NBLK * 128
_SROWS = _V // 2

_mesh = lambda: plsc.VectorSubcoreMesh(core_axis_name="c", subcore_axis_name="s")
_params = lambda: pltpu.CompilerParams(
    use_tc_tiling_on_sc=True, needs_layout_passes=False
)


def _wid():
    return lax.axis_index("s") * _NC + lax.axis_index("c")


def _iota16():
    return lax.iota(jnp.int32, 16)


@functools.partial(
    pl.kernel,
    out_type=jax.ShapeDtypeStruct((_SROWS, 128), jnp.float32),
    mesh=_mesh(),
    scratch_types=[
        [pltpu.VMEM((_D, 128), jnp.float32) for _ in range(2)],
        pltpu.VMEM((_D, 128), jnp.float32),
        pltpu.VMEM((32, 128), jnp.float32),
        pltpu.SemaphoreType.DMA,
    ],
    compiler_params=_params(),
)
def _pack_table(tt, tail, scratch, stg, tbuf, tailv, gsem):
    wid = _wid()
    scale = jnp.float32(_SCALE)
    base = wid * 244 + jnp.minimum(wid, 4)
    nblk = 244 + jnp.where(wid < 4, 1, 0)

    row_idx = [(_iota16() + 16 * (k % 4)) for k in range(8)]

    def fire(i, bi):
        b = base + i
        pltpu.async_copy(
            tt.at[pl.ds(0, _D), pl.ds(b * 128, 128)], stg[bi], gsem
        )

    def drain(bi):
        pltpu.make_async_copy(
            tt.at[pl.ds(0, _D), pl.ds(0, 128)], stg[bi], gsem
        ).wait()

    fire(0, 0)

    @pl.loop(0, nblk, step=2)
    def _blk(i0):
        for sub in range(2):
            i = i0 + sub
            @pl.when(i < nblk)
            def _do():
                drain(sub)

                @pl.when(i + 1 < nblk)
                def _pre():
                    fire(i + 1, 1 - sub)

                @plsc.parallel_loop(0, _D, unroll=4)
                def _tr(s):
                    for k in range(8):
                        col = jnp.broadcast_to(
                            (2 * s + (1 if k >= 4 else 0)).astype(jnp.int32), (16,)
                        )
                        vals = plsc.load_gather(stg[sub], [row_idx[k], col])
                        tbuf[s, pl.ds(16 * k, 16)] = vals * scale

                pltpu.sync_copy(
                    tbuf, scratch.at[pl.ds((base + i) * 64, 64), pl.ds(0, 128)]
                )

    @pl.when(wid == _NW - 1)
    def _tail():
        pltpu.sync_copy(tail, tailv)
        pltpu.sync_copy(tailv, scratch.at[pl.ds(_VTAIL // 2, 32), pl.ds(0, 128)])


@functools.partial(
    pl.kernel,
    out_type=jax.ShapeDtypeStruct((50, _D, 16384), jnp.float32),
    mesh=_mesh(),
    scratch_types=[
        pltpu.VMEM((8, 128), jnp.int32),
        pltpu.VMEM((8, 128), jnp.int32),
        [pltpu.VMEM((128, 128), jnp.float32) for _ in range(2)],
        pltpu.VMEM((_D, 128), jnp.float32),
        pltpu.SemaphoreType.DMA,
    ],
    compiler_params=_params(),
)
def _emb(xt, scratch, out, idx_v, sidx, rowb, tbuf, gsem):
    wid = _wid()
    i16 = _iota16()
    row_idx = [(i16 + 16 * k) for k in range(8)]

    def fire(hh, bi):
        pltpu.async_copy(scratch.at[sidx.at[hh]], rowb[bi], gsem)

    def drain(bi):
        pltpu.make_async_copy(
            scratch.at[pl.ds(0, 128), pl.ds(0, 128)], rowb[bi], gsem
        ).wait()

    @pl.loop(0, 28)
    def _unit(u):
        h8 = u >> 2
        bb = wid * 4 + (u & 3)
        hmax = jnp.minimum(8, 50 - 8 * h8)

        pltpu.sync_copy(
            xt.at[pl.ds(h8 * 8, 8), pl.ds(bb * 128, 128)], idx_v
        )
        for r in range(8):
            for k in range(8):
                sl = pl.ds(16 * k, 16)
                sidx[r, sl] = lax.shift_right_logical(idx_v[r, sl], 1)

        fire(0, 0)

        @pl.loop(0, hmax, step=2)
        def _h(h0):
            for sub in range(2):
                hh = h0 + sub
                drain(sub)

                @pl.when(hh + 1 < hmax)
                def _pre():
                    fire(hh + 1, 1 - sub)

                pk = [
                    lax.shift_left(
                        lax.bitwise_and(idx_v[hh, pl.ds(16 * k, 16)], 1), 6
                    )
                    for k in range(8)
                ]

                @plsc.parallel_loop(0, _D, unroll=4)
                def _tr(f):
                    for k in range(8):
                        vals = plsc.load_gather(
                            rowb[sub], [row_idx[k], pk[k] + f.astype(jnp.int32)]
                        )
                        tbuf[f, pl.ds(16 * k, 16)] = vals

                pltpu.sync_copy(
                    tbuf,
                    out.at[h8 * 8 + hh, pl.ds(0, _D), pl.ds(bb * 128, 128)],
                )


@jax.jit
def _run(x, table):
    tt = table.T
    tail = table[_VTAIL:, :].reshape(32, 128) * jnp.float32(_SCALE)
    xt = jnp.pad(x.astype(jnp.int32).T, ((0, 6), (0, 0)))
    scratch = _pack_table(tt, tail)
    out = _emb(xt, scratch)
    return out.transpose(2, 0, 1)


def kernel(x, table):
    assert x.shape == (16384, 50) and table.shape == (_V, _D)
    return _run(x, table)


# speedup vs baseline: 1.1024x; 1.1024x over previous
import functools
import math

import jax
import jax.numpy as jnp
from jax import lax
from jax.experimental import pallas as pl
from jax.experimental.pallas import tpu as pltpu
from jax.experimental.pallas import tpu_sc as plsc

_INFO = plsc.get_sparse_core_info()
_NC = _INFO.num_cores
_NS = _INFO.num_subcores
_NW = _NC * _NS
_L = _INFO.num_lanes

_V = 1000000
_D = 64
_SCALE = math.sqrt(_D)
_NBLK = (_V // 128)
_VTAIL = _NBLK * 128
_SROWS = _V // 2

_mesh = lambda: plsc.VectorSubcoreMesh(core_axis_name="c", subcore_axis_name="s")
_params = lambda: pltpu.CompilerParams(
    use_tc_tiling_on_sc=True, needs_layout_passes=False
)


def _wid():
    return lax.axis_index("s") * _NC + lax.axis_index("c")


def _iota16():
    return lax.iota(jnp.int32, 16)


@functools.partial(
    pl.kernel,
    out_type=jax.ShapeDtypeStruct((_SROWS, 128), jnp.float32),
    mesh=_mesh(),
    scratch_types=[
        [pltpu.VMEM((_D, 128), jnp.float32) for _ in range(2)],
        [pltpu.VMEM((_D, 128), jnp.float32) for _ in range(2)],
        pltpu.VMEM((32, 128), jnp.float32),
        pltpu.SemaphoreType.DMA,
        [pltpu.SemaphoreType.DMA for _ in range(2)],
    ],
    compiler_params=_params(),
)
def _pack_table(tt, tail, scratch, stg, tbuf, tailv, gsem, ssems):
    wid = _wid()
    scale = jnp.float32(_SCALE)
    base = wid * 244 + jnp.minimum(wid, 4)
    nblk = 244 + jnp.where(wid < 4, 1, 0)

    row_idx = [(_iota16() + 16 * (k % 4)) for k in range(8)]

    def fire(i, bi):
        pltpu.async_copy(
            tt.at[pl.ds(0, _D), pl.ds((base + i) * 128, 128)], stg[bi], gsem
        )

    def drain(bi):
        pltpu.make_async_copy(
            tt.at[pl.ds(0, _D), pl.ds(0, 128)], stg[bi], gsem
        ).wait()

    def drain_store(bi):
        pltpu.make_async_copy(
            tbuf[bi], scratch.at[pl.ds(0, _D), pl.ds(0, 128)], ssems[bi]
        ).wait()

    fire(0, 0)

    @pl.loop(0, nblk, step=2)
    def _blk(i0):
        for sub in range(2):
            i = i0 + sub
            @pl.when(i < nblk)
            def _do():
                drain(sub)

                @pl.when(i + 1 < nblk)
                def _pre():
                    fire(i + 1, 1 - sub)

                @pl.when(i >= 2)
                def _free():
                    drain_store(sub)

                @plsc.parallel_loop(0, _D, unroll=8)
                def _tr(s):
                    col0 = jnp.broadcast_to((2 * s).astype(jnp.int32), (16,))
                    col1 = col0 + 1
                    for k in range(8):
                        vals = plsc.load_gather(
                            stg[sub], [row_idx[k], col0 if k < 4 else col1]
                        )
                        tbuf[sub][s, pl.ds(16 * k, 16)] = vals * scale

                pltpu.async_copy(
                    tbuf[sub],
                    scratch.at[pl.ds((base + i) * 64, 64), pl.ds(0, 128)],
                    ssems[sub],
                )

    @pl.when(nblk >= 1)
    def _d0():
        drain_store(0)

    @pl.when(nblk >= 2)
    def _d1():
        drain_store(1)

    @pl.when(wid == _NW - 1)
    def _tail():
        pltpu.sync_copy(tail, tailv)
        pltpu.sync_copy(tailv, scratch.at[pl.ds(_VTAIL // 2, 32), pl.ds(0, 128)])


@functools.partial(
    pl.kernel,
    out_type=jax.ShapeDtypeStruct((50, _D, 16384), jnp.float32),
    mesh=_mesh(),
    scratch_types=[
        pltpu.VMEM((8, 128), jnp.int32),
        pltpu.VMEM((8, 128), jnp.int32),
        [pltpu.VMEM((128, 128), jnp.float32) for _ in range(2)],
        [pltpu.VMEM((_D, 128), jnp.float32) for _ in range(2)],
        pltpu.SemaphoreType.DMA,
        [pltpu.SemaphoreType.DMA for _ in range(2)],
    ],
    compiler_params=_params(),
)
def _emb(xt, scratch, out, idx_v, sidx, rowb, tbuf, gsem, ssems):
    wid = _wid()
    i16 = _iota16()
    row_idx = [(i16 + 16 * k) for k in range(8)]

    def fire(hh, bi):
        pltpu.async_copy(scratch.at[sidx.at[hh]], rowb[bi], gsem)

    def drain(bi):
        pltpu.make_async_copy(
            scratch.at[pl.ds(0, 128), pl.ds(0, 128)], rowb[bi], gsem
        ).wait()

    def drain_store(bi):
        pltpu.make_async_copy(
            tbuf[bi],
            out.at[0, pl.ds(0, _D), pl.ds(0, 128)],
            ssems[bi],
        ).wait()

    @pl.loop(0, 28, init_carry=jnp.int32(0))
    def _unit(u, nstores):
        h8 = u >> 2
        bb = wid * 4 + (u & 3)
        hmax = jnp.minimum(8, 50 - 8 * h8)

        pltpu.sync_copy(
            xt.at[pl.ds(h8 * 8, 8), pl.ds(bb * 128, 128)], idx_v
        )
        for r in range(8):
            for k in range(8):
                sl = pl.ds(16 * k, 16)
                sidx[r, sl] = lax.shift_right_logical(idx_v[r, sl], 1)

        fire(0, 0)

        @pl.loop(0, hmax, step=2, init_carry=nstores)
        def _h(h0, ns):
            for sub in range(2):
                hh = h0 + sub
                drain(sub)

                @pl.when(hh + 1 < hmax)
                def _pre():
                    fire(hh + 1, 1 - sub)

                ns = ns + 1

                @pl.when(ns > 2)
                def _free():
                    drain_store(sub)

                pk = [
                    lax.shift_left(
                        lax.bitwise_and(idx_v[hh, pl.ds(16 * k, 16)], 1), 6
                    )
                    for k in range(8)
                ]

                @plsc.parallel_loop(0, _D, unroll=8)
                def _tr(f):
                    fb = jnp.broadcast_to(f.astype(jnp.int32), (16,))
                    for k in range(8):
                        vals = plsc.load_gather(rowb[sub], [row_idx[k], pk[k] + fb])
                        tbuf[sub][f, pl.ds(16 * k, 16)] = vals

                pltpu.async_copy(
                    tbuf[sub],
                    out.at[h8 * 8 + hh, pl.ds(0, _D), pl.ds(bb * 128, 128)],
                    ssems[sub],
                )
            return ns

        return nstores + hmax

    drain_store(0)
    drain_store(1)


@jax.jit
def _run(x, table):
    tt = table.T
    tail = table[_VTAIL:, :].reshape(32, 128) * jnp.float32(_SCALE)
    xt = jnp.pad(x.astype(jnp.int32).T, ((0, 6), (0, 0)))
    scratch = _pack_table(tt, tail)
    out = _emb(xt, scratch)
    return out.transpose(2, 0, 1)


def kernel(x, table):
    assert x.shape == (16384, 50) and table.shape == (_V, _D)
    return _run(x, table)
